# Initial kernel scaffold; baseline (speedup 1.0000x reference)
import functools

import jax
import jax.numpy as jnp
from jax import lax
from jax.experimental import pallas as pl
from jax.experimental.pallas import tpu as pltpu
from jax.experimental.pallas import tpu_sc as plsc

BATCH = 16384
ENT_D = 128
REL_D = 64
NUM_BUCKETS = 52
SECONDS_PER_WEEK = 7 * 86400

NUM_CORES = 2
NUM_SUBCORES = 16
NUM_WORKERS = NUM_CORES * NUM_SUBCORES
ROWS_PER_WORKER = BATCH // NUM_WORKERS
CHUNK = 128
NUM_CHUNKS = ROWS_PER_WORKER // CHUNK

TC_BLOCK = 256
TC_GRID = BATCH // TC_BLOCK


def _sc_gather(entity_table, relation_table, head_idx, tail_idx, relation_idx):
    mesh = plsc.VectorSubcoreMesh(core_axis_name="c", subcore_axis_name="s")

    @functools.partial(
        pl.kernel,
        mesh=mesh,
        out_type=(
            jax.ShapeDtypeStruct((BATCH, ENT_D), jnp.float32),
            jax.ShapeDtypeStruct((BATCH, ENT_D), jnp.float32),
            jax.ShapeDtypeStruct((BATCH, REL_D), jnp.float32),
        ),
        scratch_types=[
            pltpu.VMEM((CHUNK,), jnp.int32),
            pltpu.VMEM((CHUNK,), jnp.int32),
            pltpu.VMEM((CHUNK,), jnp.int32),
            pltpu.VMEM((CHUNK, ENT_D), jnp.float32),
            pltpu.VMEM((CHUNK, ENT_D), jnp.float32),
            pltpu.VMEM((CHUNK, REL_D), jnp.float32),
            pltpu.SemaphoreType.DMA,
        ],
    )
    def gather_kernel(ent_hbm, rel_hbm, hidx_hbm, tidx_hbm, ridx_hbm,
                      out_h, out_t, out_r,
                      hidx_v, tidx_v, ridx_v, h_v, t_v, r_v, sem):
        wid = lax.axis_index("s") * NUM_CORES + lax.axis_index("c")
        for g in range(NUM_CHUNKS):
            base = wid * ROWS_PER_WORKER + g * CHUNK
            pltpu.sync_copy(hidx_hbm.at[pl.ds(base, CHUNK)], hidx_v)
            pltpu.sync_copy(tidx_hbm.at[pl.ds(base, CHUNK)], tidx_v)
            pltpu.sync_copy(ridx_hbm.at[pl.ds(base, CHUNK)], ridx_v)
            ch = pltpu.async_copy(ent_hbm.at[hidx_v], h_v, sem)
            ct = pltpu.async_copy(ent_hbm.at[tidx_v], t_v, sem)
            cr = pltpu.async_copy(rel_hbm.at[ridx_v], r_v, sem)
            ch.wait()
            ct.wait()
            cr.wait()
            pltpu.sync_copy(h_v, out_h.at[pl.ds(base, CHUNK)])
            pltpu.sync_copy(t_v, out_t.at[pl.ds(base, CHUNK)])
            pltpu.sync_copy(r_v, out_r.at[pl.ds(base, CHUNK)])

    return gather_kernel(entity_table, relation_table,
                         head_idx, tail_idx, relation_idx)


def _tc_body(ts_ref, tn_ref, h_ref, t_ref, r_ref, o_ref):
    ts = ts_ref[0, 0, :]
    buckets = jnp.minimum(ts // SECONDS_PER_WEEK, NUM_BUCKETS - 1)
    onehot = (buckets[:, None]
              == lax.broadcasted_iota(jnp.int32, (TC_BLOCK, NUM_BUCKETS), 1)
              ).astype(jnp.float32)
    normals = jnp.dot(onehot, tn_ref[...],
                      preferred_element_type=jnp.float32)
    h = h_ref[...]
    t = t_ref[...]
    r = r_ref[...]
    dp_h = jnp.sum(h * normals, axis=-1, keepdims=True)
    h_p = h - dp_h * normals
    dp_t = jnp.sum(t * normals, axis=-1, keepdims=True)
    t_p = t - dp_t * normals
    re_h = h_p[:, :REL_D]
    im_h = h_p[:, REL_D:]
    re_t = t_p[:, :REL_D]
    im_t = t_p[:, REL_D:]
    re_r = jnp.cos(r)
    im_r = jnp.sin(r)
    re_s = re_h * re_r - im_h * im_r - re_t
    im_s = re_h * im_r + im_h * re_r - im_t
    o_ref[...] = -jnp.sum(jnp.sqrt(re_s * re_s + im_s * im_s + 1e-12), axis=-1)


def _tc_compute(h_rows, t_rows, r_rows, timestamps, time_normals):
    ts3 = timestamps.astype(jnp.int32).reshape(TC_GRID, 1, TC_BLOCK)
    return pl.pallas_call(
        _tc_body,
        grid=(TC_GRID,),
        in_specs=[
            pl.BlockSpec((1, 1, TC_BLOCK), lambda i: (i, 0, 0)),
            pl.BlockSpec((NUM_BUCKETS, ENT_D), lambda i: (0, 0)),
            pl.BlockSpec((TC_BLOCK, ENT_D), lambda i: (i, 0)),
            pl.BlockSpec((TC_BLOCK, ENT_D), lambda i: (i, 0)),
            pl.BlockSpec((TC_BLOCK, REL_D), lambda i: (i, 0)),
        ],
        out_specs=pl.BlockSpec((TC_BLOCK,), lambda i: (i,)),
        out_shape=jax.ShapeDtypeStruct((BATCH,), jnp.float32),
    )(ts3, time_normals, h_rows, t_rows, r_rows)


def kernel(head_idx, relation_idx, tail_idx, timestamps,
           entity_table, relation_table, time_normals):
    h_rows, t_rows, r_rows = _sc_gather(
        entity_table, relation_table,
        head_idx.astype(jnp.int32), tail_idx.astype(jnp.int32),
        relation_idx.astype(jnp.int32))
    return _tc_compute(h_rows, t_rows, r_rows, timestamps, time_normals)


# speedup vs baseline: 2.0352x; 2.0352x over previous
import functools

import jax
import jax.numpy as jnp
from jax import lax
from jax.experimental import pallas as pl
from jax.experimental.pallas import tpu as pltpu
from jax.experimental.pallas import tpu_sc as plsc

BATCH = 16384
ENT_D = 128
REL_D = 64
NUM_BUCKETS = 52
SECONDS_PER_WEEK = 7 * 86400

NUM_CORES = 2
NUM_SUBCORES = 16
NUM_WORKERS = NUM_CORES * NUM_SUBCORES
ROWS_PER_WORKER = BATCH // NUM_WORKERS
CHUNK = 128
NUM_CHUNKS = ROWS_PER_WORKER // CHUNK

TC_BLOCK = 256
TC_GRID = BATCH // TC_BLOCK


def _rel_cos_sin(relation_table):
    def body(r_ref, o_ref):
        r = r_ref[...]
        o_ref[:, :REL_D] = jnp.cos(r)
        o_ref[:, REL_D:] = jnp.sin(r)

    return pl.pallas_call(
        body,
        out_shape=jax.ShapeDtypeStruct((relation_table.shape[0], ENT_D),
                                       jnp.float32),
    )(relation_table)


def _sc_gather(entity_table, rel_cs_table, head_idx, tail_idx, relation_idx):
    mesh = plsc.VectorSubcoreMesh(core_axis_name="c", subcore_axis_name="s")

    @functools.partial(
        pl.kernel,
        mesh=mesh,
        out_type=(
            jax.ShapeDtypeStruct((BATCH, ENT_D), jnp.float32),
            jax.ShapeDtypeStruct((BATCH, ENT_D), jnp.float32),
            jax.ShapeDtypeStruct((BATCH, ENT_D), jnp.float32),
        ),
        scratch_types=[
            pltpu.VMEM((CHUNK,), jnp.int32),
            pltpu.VMEM((CHUNK,), jnp.int32),
            pltpu.VMEM((CHUNK,), jnp.int32),
            pltpu.VMEM((CHUNK, ENT_D), jnp.float32),
            pltpu.VMEM((CHUNK, ENT_D), jnp.float32),
            pltpu.VMEM((CHUNK, ENT_D), jnp.float32),
            pltpu.SemaphoreType.DMA,
        ],
    )
    def gather_kernel(ent_hbm, rel_hbm, hidx_hbm, tidx_hbm, ridx_hbm,
                      out_h, out_t, out_r,
                      hidx_v, tidx_v, ridx_v, h_v, t_v, r_v, sem):
        wid = lax.axis_index("s") * NUM_CORES + lax.axis_index("c")
        for g in range(NUM_CHUNKS):
            base = wid * ROWS_PER_WORKER + g * CHUNK
            pltpu.sync_copy(hidx_hbm.at[pl.ds(base, CHUNK)], hidx_v)
            pltpu.sync_copy(tidx_hbm.at[pl.ds(base, CHUNK)], tidx_v)
            pltpu.sync_copy(ridx_hbm.at[pl.ds(base, CHUNK)], ridx_v)
            ch = pltpu.async_copy(ent_hbm.at[hidx_v], h_v, sem)
            ct = pltpu.async_copy(ent_hbm.at[tidx_v], t_v, sem)
            cr = pltpu.async_copy(rel_hbm.at[ridx_v], r_v, sem)
            ch.wait()
            ct.wait()
            cr.wait()
            pltpu.sync_copy(h_v, out_h.at[pl.ds(base, CHUNK)])
            pltpu.sync_copy(t_v, out_t.at[pl.ds(base, CHUNK)])
            pltpu.sync_copy(r_v, out_r.at[pl.ds(base, CHUNK)])

    return gather_kernel(entity_table, rel_cs_table,
                         head_idx, tail_idx, relation_idx)


def _tc_body(ts_ref, tn_ref, h_ref, t_ref, r_ref, o_ref):
    ts = ts_ref[0, 0, :]
    buckets = jnp.minimum(ts // SECONDS_PER_WEEK, NUM_BUCKETS - 1)
    onehot = (buckets[:, None]
              == lax.broadcasted_iota(jnp.int32, (TC_BLOCK, NUM_BUCKETS), 1)
              ).astype(jnp.float32)
    normals = jnp.dot(onehot, tn_ref[...],
                      preferred_element_type=jnp.float32)
    h = h_ref[...]
    t = t_ref[...]
    dp_h = jnp.sum(h * normals, axis=-1, keepdims=True)
    h_p = h - dp_h * normals
    dp_t = jnp.sum(t * normals, axis=-1, keepdims=True)
    t_p = t - dp_t * normals
    re_h = h_p[:, :REL_D]
    im_h = h_p[:, REL_D:]
    re_t = t_p[:, :REL_D]
    im_t = t_p[:, REL_D:]
    re_r = r_ref[:, :REL_D]
    im_r = r_ref[:, REL_D:]
    re_s = re_h * re_r - im_h * im_r - re_t
    im_s = re_h * im_r + im_h * re_r - im_t
    o_ref[...] = -jnp.sum(jnp.sqrt(re_s * re_s + im_s * im_s + 1e-12), axis=-1)


def _tc_compute(h_rows, t_rows, r_rows, timestamps, time_normals):
    ts3 = timestamps.astype(jnp.int32).reshape(TC_GRID, 1, TC_BLOCK)
    return pl.pallas_call(
        _tc_body,
        grid=(TC_GRID,),
        in_specs=[
            pl.BlockSpec((1, 1, TC_BLOCK), lambda i: (i, 0, 0)),
            pl.BlockSpec((NUM_BUCKETS, ENT_D), lambda i: (0, 0)),
            pl.BlockSpec((TC_BLOCK, ENT_D), lambda i: (i, 0)),
            pl.BlockSpec((TC_BLOCK, ENT_D), lambda i: (i, 0)),
            pl.BlockSpec((TC_BLOCK, ENT_D), lambda i: (i, 0)),
        ],
        out_specs=pl.BlockSpec((TC_BLOCK,), lambda i: (i,)),
        out_shape=jax.ShapeDtypeStruct((BATCH,), jnp.float32),
    )(ts3, time_normals, h_rows, t_rows, r_rows)


def kernel(head_idx, relation_idx, tail_idx, timestamps,
           entity_table, relation_table, time_normals):
    rel_cs = _rel_cos_sin(relation_table)
    h_rows, t_rows, r_rows = _sc_gather(
        entity_table, rel_cs,
        head_idx.astype(jnp.int32), tail_idx.astype(jnp.int32),
        relation_idx.astype(jnp.int32))
    return _tc_compute(h_rows, t_rows, r_rows, timestamps, time_normals)


# speedup vs baseline: 2.5972x; 1.2761x over previous
import functools

import jax
import jax.numpy as jnp
from jax import lax
from jax.experimental import pallas as pl
from jax.experimental.pallas import tpu as pltpu
from jax.experimental.pallas import tpu_sc as plsc

BATCH = 16384
ENT_D = 128
REL_D = 64
NUM_BUCKETS = 52
SECONDS_PER_WEEK = 7 * 86400

NUM_CORES = 2
NUM_SUBCORES = 16
NUM_WORKERS = NUM_CORES * NUM_SUBCORES
ROWS_PER_WORKER = BATCH // NUM_WORKERS
CHUNK = 128
NUM_CHUNKS = ROWS_PER_WORKER // CHUNK

TC_BLOCK = 1024
TC_GRID = BATCH // TC_BLOCK


def _rel_cos_sin(relation_table):
    def body(r_ref, o_ref):
        r = r_ref[...]
        o_ref[:, :REL_D] = jnp.cos(r)
        o_ref[:, REL_D:] = jnp.sin(r)

    return pl.pallas_call(
        body,
        out_shape=jax.ShapeDtypeStruct((relation_table.shape[0], ENT_D),
                                       jnp.float32),
    )(relation_table)


def _sc_gather(entity_table, rel_cs_table, head_idx, tail_idx, relation_idx):
    mesh = plsc.VectorSubcoreMesh(core_axis_name="c", subcore_axis_name="s")

    @functools.partial(
        pl.kernel,
        mesh=mesh,
        out_type=(
            jax.ShapeDtypeStruct((BATCH, ENT_D), jnp.float32),
            jax.ShapeDtypeStruct((BATCH, ENT_D), jnp.float32),
            jax.ShapeDtypeStruct((BATCH, ENT_D), jnp.float32),
        ),
        scratch_types=[
            pltpu.VMEM((CHUNK,), jnp.int32),
            pltpu.VMEM((CHUNK,), jnp.int32),
            pltpu.VMEM((CHUNK,), jnp.int32),
            pltpu.VMEM((CHUNK, ENT_D), jnp.float32),
            pltpu.VMEM((CHUNK, ENT_D), jnp.float32),
            pltpu.VMEM((CHUNK, ENT_D), jnp.float32),
            pltpu.SemaphoreType.DMA,
        ],
    )
    def gather_kernel(ent_hbm, rel_hbm, hidx_hbm, tidx_hbm, ridx_hbm,
                      out_h, out_t, out_r,
                      hidx_v, tidx_v, ridx_v, h_v, t_v, r_v, sem):
        wid = lax.axis_index("s") * NUM_CORES + lax.axis_index("c")
        for g in range(NUM_CHUNKS):
            base = wid * ROWS_PER_WORKER + g * CHUNK
            pltpu.sync_copy(hidx_hbm.at[pl.ds(base, CHUNK)], hidx_v)
            pltpu.sync_copy(tidx_hbm.at[pl.ds(base, CHUNK)], tidx_v)
            pltpu.sync_copy(ridx_hbm.at[pl.ds(base, CHUNK)], ridx_v)
            ch = pltpu.async_copy(ent_hbm.at[hidx_v], h_v, sem)
            ct = pltpu.async_copy(ent_hbm.at[tidx_v], t_v, sem)
            cr = pltpu.async_copy(rel_hbm.at[ridx_v], r_v, sem)
            ch.wait()
            ct.wait()
            cr.wait()
            pltpu.sync_copy(h_v, out_h.at[pl.ds(base, CHUNK)])
            pltpu.sync_copy(t_v, out_t.at[pl.ds(base, CHUNK)])
            pltpu.sync_copy(r_v, out_r.at[pl.ds(base, CHUNK)])

    return gather_kernel(entity_table, rel_cs_table,
                         head_idx, tail_idx, relation_idx)


def _tc_body(ts_ref, tn_ref, h_ref, t_ref, r_ref, o_ref):
    ts = ts_ref[0, 0, :]
    buckets = jnp.minimum(ts // SECONDS_PER_WEEK, NUM_BUCKETS - 1)
    onehot = (buckets[:, None]
              == lax.broadcasted_iota(jnp.int32, (TC_BLOCK, NUM_BUCKETS), 1)
              ).astype(jnp.float32)
    normals = jnp.dot(onehot, tn_ref[...],
                      preferred_element_type=jnp.float32)
    h = h_ref[...]
    t = t_ref[...]
    ones = jnp.ones((ENT_D, 8), jnp.float32)
    dp_h = jnp.dot(h * normals, ones,
                   preferred_element_type=jnp.float32)[:, :1]
    h_p = h - dp_h * normals
    dp_t = jnp.dot(t * normals, ones,
                   preferred_element_type=jnp.float32)[:, :1]
    t_p = t - dp_t * normals
    re_h = h_p[:, :REL_D]
    im_h = h_p[:, REL_D:]
    re_t = t_p[:, :REL_D]
    im_t = t_p[:, REL_D:]
    re_r = r_ref[:, :REL_D]
    im_r = r_ref[:, REL_D:]
    re_s = re_h * re_r - im_h * im_r - re_t
    im_s = re_h * im_r + im_h * re_r - im_t
    o_ref[...] = -jnp.sum(jnp.sqrt(re_s * re_s + im_s * im_s + 1e-12), axis=-1)


def _tc_compute(h_rows, t_rows, r_rows, timestamps, time_normals):
    ts3 = timestamps.astype(jnp.int32).reshape(TC_GRID, 1, TC_BLOCK)
    return pl.pallas_call(
        _tc_body,
        grid=(TC_GRID,),
        in_specs=[
            pl.BlockSpec((1, 1, TC_BLOCK), lambda i: (i, 0, 0)),
            pl.BlockSpec((NUM_BUCKETS, ENT_D), lambda i: (0, 0)),
            pl.BlockSpec((TC_BLOCK, ENT_D), lambda i: (i, 0)),
            pl.BlockSpec((TC_BLOCK, ENT_D), lambda i: (i, 0)),
            pl.BlockSpec((TC_BLOCK, ENT_D), lambda i: (i, 0)),
        ],
        out_specs=pl.BlockSpec((TC_BLOCK,), lambda i: (i,)),
        out_shape=jax.ShapeDtypeStruct((BATCH,), jnp.float32),
    )(ts3, time_normals, h_rows, t_rows, r_rows)


def kernel(head_idx, relation_idx, tail_idx, timestamps,
           entity_table, relation_table, time_normals):
    rel_cs = _rel_cos_sin(relation_table)
    h_rows, t_rows, r_rows = _sc_gather(
        entity_table, rel_cs,
        head_idx.astype(jnp.int32), tail_idx.astype(jnp.int32),
        relation_idx.astype(jnp.int32))
    return _tc_compute(h_rows, t_rows, r_rows, timestamps, time_normals)
